# Initial kernel scaffold; baseline (speedup 1.0000x reference)
import jax
import jax.numpy as jnp
from jax.experimental import pallas as pl


def kernel(x, edge_index, W1_l, b1, W1_r, W2_l, b2, W2_r):
    raise NotImplementedError("write your pallas kernel here")


# speedup vs baseline: 4.6495x; 4.6495x over previous
import jax
import jax.numpy as jnp
from jax import lax
from jax.experimental import pallas as pl
from jax.experimental.pallas import tpu as pltpu
from jax.experimental.pallas import tpu_sc as plsc

_NC = 2
_NS = 16
_CHUNK = 128


def _make_sc_agg(n_pad, e_pad, d):
  nw = _NC * _NS
  per_w = e_pad // nw
  n_chunks = per_w // _CHUNK
  rows_per_tile = n_pad // _NS
  mesh = plsc.VectorSubcoreMesh(core_axis_name="c", subcore_axis_name="s")

  def body(x_hbm, src_hbm, dst_hbm, z_hbm, acc_out,
           src_v, dst_v, rows_v, acc_sh, sem):
    cid = lax.axis_index("c")
    sid = lax.axis_index("s")
    wid = cid * _NS + sid
    r0 = sid * rows_per_tile

    pltpu.sync_copy(z_hbm.at[pl.ds(r0, rows_per_tile)],
                    acc_sh.at[pl.ds(r0, rows_per_tile)])
    plsc.subcore_barrier()

    base = wid * per_w

    def step(j, carry):
      off = base + j * _CHUNK
      pltpu.sync_copy(src_hbm.at[pl.ds(off, _CHUNK)], src_v)
      pltpu.sync_copy(dst_hbm.at[pl.ds(off, _CHUNK)], dst_v)
      pltpu.async_copy(x_hbm.at[src_v], rows_v, sem).wait()
      pltpu.sync_copy(rows_v, acc_sh.at[dst_v], add=True)
      return carry

    lax.fori_loop(0, n_chunks, step, 0)
    plsc.subcore_barrier()

    pltpu.sync_copy(acc_sh.at[pl.ds(r0, rows_per_tile)],
                    acc_out.at[cid, pl.ds(r0, rows_per_tile)])

  return pl.kernel(
      body,
      out_type=jax.ShapeDtypeStruct((_NC, n_pad, d), jnp.float32),
      mesh=mesh,
      scratch_types=[
          pltpu.VMEM((_CHUNK,), jnp.int32),
          pltpu.VMEM((_CHUNK,), jnp.int32),
          pltpu.VMEM((_CHUNK, d), jnp.float32),
          pltpu.VMEM_SHARED((n_pad, d), jnp.float32),
          pltpu.SemaphoreType.DMA,
      ])


def _make_sc_cnt(n_pad, e_pad):
  nw = _NC * _NS
  per_w = e_pad // nw
  n_chunks = per_w // _CHUNK
  rows_per_tile = n_pad // _NS
  d = 128
  mesh = plsc.VectorSubcoreMesh(core_axis_name="c", subcore_axis_name="s")

  def body(dst_hbm, z_hbm, ones_hbm, cnt_out, dst_v, ones_v, cnt_sh):
    cid = lax.axis_index("c")
    sid = lax.axis_index("s")
    wid = cid * _NS + sid
    r0 = sid * rows_per_tile

    pltpu.sync_copy(z_hbm.at[pl.ds(r0, rows_per_tile)],
                    cnt_sh.at[pl.ds(r0, rows_per_tile)])
    pltpu.sync_copy(ones_hbm, ones_v)
    plsc.subcore_barrier()

    base = wid * per_w

    def step(j, carry):
      off = base + j * _CHUNK
      pltpu.sync_copy(dst_hbm.at[pl.ds(off, _CHUNK)], dst_v)
      pltpu.sync_copy(ones_v, cnt_sh.at[dst_v], add=True)
      return carry

    lax.fori_loop(0, n_chunks, step, 0)
    plsc.subcore_barrier()

    pltpu.sync_copy(cnt_sh.at[pl.ds(r0, rows_per_tile)],
                    cnt_out.at[cid, pl.ds(r0, rows_per_tile)])

  return pl.kernel(
      body,
      out_type=jax.ShapeDtypeStruct((_NC, n_pad, d), jnp.float32),
      mesh=mesh,
      scratch_types=[
          pltpu.VMEM((_CHUNK,), jnp.int32),
          pltpu.VMEM((_CHUNK, d), jnp.float32),
          pltpu.VMEM_SHARED((n_pad, d), jnp.float32),
      ])


def _dot_t(a, b):
  return lax.dot_general(a, b, (((1,), (1,)), ((), ())),
                         preferred_element_type=jnp.float32)


def _tc_layer1(accp_ref, cnt_ref, x_ref, w1l_ref, b1_ref, w1r_ref,
               w2l_ref, w2r_ref, b2_ref, p_ref, q_ref, invb_ref):
  s = accp_ref[0] + accp_ref[1]
  cnt = cnt_ref[0][:, 0:1] + cnt_ref[1][:, 0:1]
  inv = 1.0 / jnp.maximum(cnt, 1.0)
  sm = s * inv
  h = _dot_t(sm, w1l_ref[...]) + b1_ref[...] + _dot_t(x_ref[...], w1r_ref[...])
  h = jnp.maximum(h, 0.0)
  p_ref[...] = _dot_t(h, w2l_ref[...])
  q_ref[...] = _dot_t(h, w2r_ref[...]) + b2_ref[...]
  invb_ref[...] = jnp.broadcast_to(inv, invb_ref.shape)


def _tc_final(accp_ref, invb_ref, q_ref, out_ref):
  out_ref[...] = (accp_ref[0] + accp_ref[1]) * invb_ref[...] + q_ref[...]


def kernel(x, edge_index, W1_l, b1, W1_r, W2_l, b2, W2_r):
  n, d_in = x.shape
  d_hid = W1_l.shape[0]
  d_out = W2_l.shape[0]
  e = edge_index.shape[1]

  nw = _NC * _NS
  n_pad = -(-(n + 1) // (_NS * 8)) * (_NS * 8)
  chunks_total = -(-e // _CHUNK)
  per_w = -(-chunks_total // nw) * _CHUNK
  e_pad = per_w * nw

  src = edge_index[0].astype(jnp.int32)
  dst = edge_index[1].astype(jnp.int32)
  src = jnp.concatenate([src, jnp.zeros((e_pad - e,), jnp.int32)])
  dst = jnp.concatenate([dst, jnp.full((e_pad - e,), n, jnp.int32)])

  acc1 = _make_sc_agg(n_pad, e_pad, d_in)(
      x, src, dst, jnp.zeros((n_pad, d_in), jnp.float32))
  cnt = _make_sc_cnt(n_pad, e_pad)(
      dst, jnp.zeros((n_pad, 128), jnp.float32),
      jnp.ones((_CHUNK, 128), jnp.float32))

  blk = 1000
  grid = (n // blk,)
  full = lambda shape: pl.BlockSpec(shape, lambda i: (0,) * len(shape))
  p, q, invb = pl.pallas_call(
      _tc_layer1,
      grid=grid,
      in_specs=[
          pl.BlockSpec((_NC, blk, d_in), lambda i: (0, i, 0)),
          pl.BlockSpec((_NC, blk, 128), lambda i: (0, i, 0)),
          pl.BlockSpec((blk, d_in), lambda i: (i, 0)),
          full((d_hid, d_in)),
          full((1, d_hid)),
          full((d_hid, d_in)),
          full((d_out, d_hid)),
          full((d_out, d_hid)),
          full((1, d_out)),
      ],
      out_specs=[
          pl.BlockSpec((blk, d_out), lambda i: (i, 0)),
          pl.BlockSpec((blk, d_out), lambda i: (i, 0)),
          pl.BlockSpec((blk, d_out), lambda i: (i, 0)),
      ],
      out_shape=[
          jax.ShapeDtypeStruct((n, d_out), jnp.float32),
          jax.ShapeDtypeStruct((n, d_out), jnp.float32),
          jax.ShapeDtypeStruct((n, d_out), jnp.float32),
      ],
  )(acc1, cnt, x, W1_l, b1.reshape(1, -1), W1_r, W2_l, W2_r, b2.reshape(1, -1))

  acc2 = _make_sc_agg(n_pad, e_pad, d_out)(
      p, src, dst, jnp.zeros((n_pad, d_out), jnp.float32))

  out = pl.pallas_call(
      _tc_final,
      grid=grid,
      in_specs=[
          pl.BlockSpec((_NC, blk, d_out), lambda i: (0, i, 0)),
          pl.BlockSpec((blk, d_out), lambda i: (i, 0)),
          pl.BlockSpec((blk, d_out), lambda i: (i, 0)),
      ],
      out_specs=pl.BlockSpec((blk, d_out), lambda i: (i, 0)),
      out_shape=jax.ShapeDtypeStruct((n, d_out), jnp.float32),
  )(acc2, invb, q)
  return out


# speedup vs baseline: 4.8033x; 1.0331x over previous
import jax
import jax.numpy as jnp
from jax import lax
from jax.experimental import pallas as pl
from jax.experimental.pallas import tpu as pltpu
from jax.experimental.pallas import tpu_sc as plsc

_NC = 2
_NS = 16
_CHUNK = 112


def _make_sc_agg(n_pad, e_pad, d):
  nw = _NC * _NS
  per_w = e_pad // nw
  n_chunks = per_w // _CHUNK
  assert n_chunks % 2 == 1
  rows_per_tile = n_pad // _NS
  mesh = plsc.VectorSubcoreMesh(core_axis_name="c", subcore_axis_name="s")

  def body(x_hbm, src_hbm, dst_hbm, z_hbm, acc_out,
           src_v, dst_v, rows_a, rows_b, acc_sh, sem_a, sem_b):
    cid = lax.axis_index("c")
    sid = lax.axis_index("s")
    wid = cid * _NS + sid
    r0 = sid * rows_per_tile
    base = wid * per_w

    pltpu.sync_copy(z_hbm.at[pl.ds(r0, rows_per_tile)],
                    acc_sh.at[pl.ds(r0, rows_per_tile)])
    pltpu.sync_copy(src_hbm.at[pl.ds(base, per_w)], src_v)
    pltpu.sync_copy(dst_hbm.at[pl.ds(base, per_w)], dst_v)
    plsc.subcore_barrier()

    def gather(c, rows, sem):
      pltpu.async_copy(x_hbm.at[src_v.at[pl.ds(c * _CHUNK, _CHUNK)]], rows, sem)

    def drain(rows, sem):
      pltpu.make_async_copy(x_hbm.at[pl.ds(0, _CHUNK)], rows, sem).wait()

    def scatter(c, rows):
      pltpu.sync_copy(rows, acc_sh.at[dst_v.at[pl.ds(c * _CHUNK, _CHUNK)]],
                      add=True)

    gather(0, rows_a, sem_a)

    def pair(i, carry):
      c = 2 * i
      gather(c + 1, rows_b, sem_b)
      drain(rows_a, sem_a)
      scatter(c, rows_a)
      gather(c + 2, rows_a, sem_a)
      drain(rows_b, sem_b)
      scatter(c + 1, rows_b)
      return carry

    lax.fori_loop(0, (n_chunks - 1) // 2, pair, 0)
    drain(rows_a, sem_a)
    scatter(n_chunks - 1, rows_a)
    plsc.subcore_barrier()

    pltpu.sync_copy(acc_sh.at[pl.ds(r0, rows_per_tile)],
                    acc_out.at[cid, pl.ds(r0, rows_per_tile)])

  return pl.kernel(
      body,
      out_type=jax.ShapeDtypeStruct((_NC, n_pad, d), jnp.float32),
      mesh=mesh,
      scratch_types=[
          pltpu.VMEM((per_w,), jnp.int32),
          pltpu.VMEM((per_w,), jnp.int32),
          pltpu.VMEM((_CHUNK, d), jnp.float32),
          pltpu.VMEM((_CHUNK, d), jnp.float32),
          pltpu.VMEM_SHARED((n_pad, d), jnp.float32),
          pltpu.SemaphoreType.DMA,
          pltpu.SemaphoreType.DMA,
      ])


def _make_sc_cnt(n_pad, e_pad):
  nw = _NC * _NS
  per_w = e_pad // nw
  n_chunks = per_w // _CHUNK
  rows_per_tile = n_pad // _NS
  d = 128
  mesh = plsc.VectorSubcoreMesh(core_axis_name="c", subcore_axis_name="s")

  def body(dst_hbm, z_hbm, ones_hbm, cnt_out, dst_v, ones_v, cnt_sh):
    cid = lax.axis_index("c")
    sid = lax.axis_index("s")
    wid = cid * _NS + sid
    r0 = sid * rows_per_tile
    base = wid * per_w

    pltpu.sync_copy(z_hbm.at[pl.ds(r0, rows_per_tile)],
                    cnt_sh.at[pl.ds(r0, rows_per_tile)])
    pltpu.sync_copy(ones_hbm, ones_v)
    pltpu.sync_copy(dst_hbm.at[pl.ds(base, per_w)], dst_v)
    plsc.subcore_barrier()

    def step(j, carry):
      pltpu.sync_copy(ones_v, cnt_sh.at[dst_v.at[pl.ds(j * _CHUNK, _CHUNK)]],
                      add=True)
      return carry

    lax.fori_loop(0, n_chunks, step, 0)
    plsc.subcore_barrier()

    pltpu.sync_copy(cnt_sh.at[pl.ds(r0, rows_per_tile)],
                    cnt_out.at[cid, pl.ds(r0, rows_per_tile)])

  return pl.kernel(
      body,
      out_type=jax.ShapeDtypeStruct((_NC, n_pad, d), jnp.float32),
      mesh=mesh,
      scratch_types=[
          pltpu.VMEM((per_w,), jnp.int32),
          pltpu.VMEM((_CHUNK, d), jnp.float32),
          pltpu.VMEM_SHARED((n_pad, d), jnp.float32),
      ])


def _dot_t(a, b):
  return lax.dot_general(a, b, (((1,), (1,)), ((), ())),
                         preferred_element_type=jnp.float32)


def _tc_layer1(accp_ref, cnt_ref, x_ref, w1l_ref, b1_ref, w1r_ref,
               w2l_ref, w2r_ref, b2_ref, p_ref, q_ref, invb_ref):
  s = accp_ref[0] + accp_ref[1]
  cnt = cnt_ref[0][:, 0:1] + cnt_ref[1][:, 0:1]
  inv = 1.0 / jnp.maximum(cnt, 1.0)
  sm = s * inv
  h = _dot_t(sm, w1l_ref[...]) + b1_ref[...] + _dot_t(x_ref[...], w1r_ref[...])
  h = jnp.maximum(h, 0.0)
  p_ref[...] = _dot_t(h, w2l_ref[...])
  q_ref[...] = _dot_t(h, w2r_ref[...]) + b2_ref[...]
  invb_ref[...] = jnp.broadcast_to(inv, invb_ref.shape)


def _tc_final(accp_ref, invb_ref, q_ref, out_ref):
  out_ref[...] = (accp_ref[0] + accp_ref[1]) * invb_ref[...] + q_ref[...]


def kernel(x, edge_index, W1_l, b1, W1_r, W2_l, b2, W2_r):
  n, d_in = x.shape
  d_hid = W1_l.shape[0]
  d_out = W2_l.shape[0]
  e = edge_index.shape[1]

  nw = _NC * _NS
  n_pad = -(-(n + 1) // (_NS * 8)) * (_NS * 8)
  cpw = -(-(-(-e // _CHUNK)) // nw)
  cpw += 1 - (cpw % 2)
  per_w = cpw * _CHUNK
  e_pad = per_w * nw

  src = edge_index[0].astype(jnp.int32)
  dst = edge_index[1].astype(jnp.int32)
  src = jnp.concatenate([src, jnp.zeros((e_pad - e,), jnp.int32)])
  dst = jnp.concatenate([dst, jnp.full((e_pad - e,), n, jnp.int32)])

  acc1 = _make_sc_agg(n_pad, e_pad, d_in)(
      x, src, dst, jnp.zeros((n_pad, d_in), jnp.float32))
  cnt = _make_sc_cnt(n_pad, e_pad)(
      dst, jnp.zeros((n_pad, 128), jnp.float32),
      jnp.ones((_CHUNK, 128), jnp.float32))

  blk = 1000
  grid = (n // blk,)
  full = lambda shape: pl.BlockSpec(shape, lambda i: (0,) * len(shape))
  p, q, invb = pl.pallas_call(
      _tc_layer1,
      grid=grid,
      in_specs=[
          pl.BlockSpec((_NC, blk, d_in), lambda i: (0, i, 0)),
          pl.BlockSpec((_NC, blk, 128), lambda i: (0, i, 0)),
          pl.BlockSpec((blk, d_in), lambda i: (i, 0)),
          full((d_hid, d_in)),
          full((1, d_hid)),
          full((d_hid, d_in)),
          full((d_out, d_hid)),
          full((d_out, d_hid)),
          full((1, d_out)),
      ],
      out_specs=[
          pl.BlockSpec((blk, d_out), lambda i: (i, 0)),
          pl.BlockSpec((blk, d_out), lambda i: (i, 0)),
          pl.BlockSpec((blk, d_out), lambda i: (i, 0)),
      ],
      out_shape=[
          jax.ShapeDtypeStruct((n, d_out), jnp.float32),
          jax.ShapeDtypeStruct((n, d_out), jnp.float32),
          jax.ShapeDtypeStruct((n, d_out), jnp.float32),
      ],
  )(acc1, cnt, x, W1_l, b1.reshape(1, -1), W1_r, W2_l, W2_r, b2.reshape(1, -1))

  acc2 = _make_sc_agg(n_pad, e_pad, d_out)(
      p, src, dst, jnp.zeros((n_pad, d_out), jnp.float32))

  out = pl.pallas_call(
      _tc_final,
      grid=grid,
      in_specs=[
          pl.BlockSpec((_NC, blk, d_out), lambda i: (0, i, 0)),
          pl.BlockSpec((blk, d_out), lambda i: (i, 0)),
          pl.BlockSpec((blk, d_out), lambda i: (i, 0)),
      ],
      out_specs=pl.BlockSpec((blk, d_out), lambda i: (i, 0)),
      out_shape=jax.ShapeDtypeStruct((n, d_out), jnp.float32),
  )(acc2, invb, q)
  return out


# speedup vs baseline: 12.6088x; 2.6250x over previous
import jax
import jax.numpy as jnp
from jax import lax
from jax.experimental import pallas as pl
from jax.experimental.pallas import tpu as pltpu
from jax.experimental.pallas import tpu_sc as plsc

_NC = 2
_NS = 16
_CHUNK = 112


def _make_sc_agg(n_pad, e_pad, d):
  nw = _NC * _NS
  per_w = e_pad // nw
  n_chunks = per_w // _CHUNK
  assert n_chunks % 2 == 1
  rows_per_tile = n_pad // _NS
  mesh = plsc.VectorSubcoreMesh(core_axis_name="c", subcore_axis_name="s")

  def body(x_hbm, src_hbm, dst_hbm, z_hbm, acc_out,
           src_v, dst_v, rows_a, rows_b, acc_sh, sem_a, sem_b):
    cid = lax.axis_index("c")
    sid = lax.axis_index("s")
    wid = cid * _NS + sid
    r0 = sid * rows_per_tile
    base = wid * per_w

    pltpu.sync_copy(z_hbm.at[pl.ds(r0, rows_per_tile)],
                    acc_sh.at[pl.ds(r0, rows_per_tile)])
    pltpu.sync_copy(src_hbm.at[pl.ds(base, per_w)], src_v)
    pltpu.sync_copy(dst_hbm.at[pl.ds(base, per_w)], dst_v)
    plsc.subcore_barrier()

    def gather(c, rows, sem):
      pltpu.async_copy(x_hbm.at[src_v.at[pl.ds(c * _CHUNK, _CHUNK)]], rows, sem)

    def drain(rows, sem):
      pltpu.make_async_copy(x_hbm.at[pl.ds(0, _CHUNK)], rows, sem).wait()

    def scatter(c, rows):
      pltpu.sync_copy(rows, acc_sh.at[dst_v.at[pl.ds(c * _CHUNK, _CHUNK)]],
                      add=True)

    gather(0, rows_a, sem_a)

    def pair(i, carry):
      c = 2 * i
      gather(c + 1, rows_b, sem_b)
      drain(rows_a, sem_a)
      scatter(c, rows_a)
      gather(c + 2, rows_a, sem_a)
      drain(rows_b, sem_b)
      scatter(c + 1, rows_b)
      return carry

    lax.fori_loop(0, (n_chunks - 1) // 2, pair, 0)
    drain(rows_a, sem_a)
    scatter(n_chunks - 1, rows_a)
    plsc.subcore_barrier()

    pltpu.sync_copy(acc_sh.at[pl.ds(r0, rows_per_tile)],
                    acc_out.at[cid, pl.ds(r0, rows_per_tile)])

  return pl.kernel(
      body,
      out_type=jax.ShapeDtypeStruct((_NC, n_pad, d), jnp.float32),
      mesh=mesh,
      scratch_types=[
          pltpu.VMEM((per_w,), jnp.int32),
          pltpu.VMEM((per_w,), jnp.int32),
          pltpu.VMEM((_CHUNK, d), jnp.float32),
          pltpu.VMEM((_CHUNK, d), jnp.float32),
          pltpu.VMEM_SHARED((n_pad, d), jnp.float32),
          pltpu.SemaphoreType.DMA,
          pltpu.SemaphoreType.DMA,
      ])


def _make_sc_cnt(n_pad, e_pad):
  nw = _NC * _NS
  per_w = e_pad // nw
  n_chunks = per_w // _CHUNK
  rows_per_tile = n_pad // _NS
  d = 128
  mesh = plsc.VectorSubcoreMesh(core_axis_name="c", subcore_axis_name="s")

  def body(dst_hbm, z_hbm, ones_hbm, cnt_out, dst_v, ones_v, cnt_sh):
    cid = lax.axis_index("c")
    sid = lax.axis_index("s")
    wid = cid * _NS + sid
    r0 = sid * rows_per_tile
    base = wid * per_w

    pltpu.sync_copy(z_hbm.at[pl.ds(r0, rows_per_tile)],
                    cnt_sh.at[pl.ds(r0, rows_per_tile)])
    pltpu.sync_copy(ones_hbm, ones_v)
    pltpu.sync_copy(dst_hbm.at[pl.ds(base, per_w)], dst_v)
    plsc.subcore_barrier()

    def step(j, carry):
      pltpu.sync_copy(ones_v, cnt_sh.at[dst_v.at[pl.ds(j * _CHUNK, _CHUNK)]],
                      add=True)
      return carry

    lax.fori_loop(0, n_chunks, step, 0)
    plsc.subcore_barrier()

    pltpu.sync_copy(cnt_sh.at[pl.ds(r0, rows_per_tile)],
                    cnt_out.at[cid, pl.ds(r0, rows_per_tile)])

  return pl.kernel(
      body,
      out_type=jax.ShapeDtypeStruct((_NC, n_pad, d), jnp.float32),
      mesh=mesh,
      scratch_types=[
          pltpu.VMEM((per_w,), jnp.int32),
          pltpu.VMEM((_CHUNK, d), jnp.float32),
          pltpu.VMEM_SHARED((n_pad, d), jnp.float32),
      ])


def _dot_t(a, b):
  return lax.dot_general(a, b, (((1,), (1,)), ((), ())),
                         preferred_element_type=jnp.float32)


def _tc_layer1(accp_ref, cnt_ref, x_ref, w1l_ref, b1_ref, w1r_ref,
               w2l_ref, w2r_ref, b2_ref, p_ref, q_ref, invb_ref):
  s = accp_ref[0] + accp_ref[1]
  cnt = cnt_ref[0][:, 0:1] + cnt_ref[1][:, 0:1]
  inv = 1.0 / jnp.maximum(cnt, 1.0)
  sm = s * inv
  h = _dot_t(sm, w1l_ref[...]) + b1_ref[...] + _dot_t(x_ref[...], w1r_ref[...])
  h = jnp.maximum(h, 0.0)
  p_ref[...] = _dot_t(h, w2l_ref[...])
  q_ref[...] = _dot_t(h, w2r_ref[...]) + b2_ref[...]
  invb_ref[...] = jnp.broadcast_to(inv, invb_ref.shape)


def _tc_final(accp_ref, invb_ref, q_ref, out_ref):
  out_ref[...] = (accp_ref[0] + accp_ref[1]) * invb_ref[...] + q_ref[...]


def kernel(x, edge_index, W1_l, b1, W1_r, W2_l, b2, W2_r):
  n, d_in = x.shape
  d_hid = W1_l.shape[0]
  d_out = W2_l.shape[0]
  e = edge_index.shape[1]

  nw = _NC * _NS
  n_pad = -(-(n + 1) // (_NS * 8)) * (_NS * 8)
  cpw = -(-(-(-e // _CHUNK)) // nw)
  cpw += 1 - (cpw % 2)
  per_w = cpw * _CHUNK
  e_pad = per_w * nw

  src = edge_index[0].astype(jnp.int32)
  dst = edge_index[1].astype(jnp.int32)
  pad_i = jnp.arange(e_pad - e, dtype=jnp.int32)
  src = jnp.concatenate([src, pad_i % n])
  dst = jnp.concatenate([dst, n + pad_i % (n_pad - n)])

  acc1 = _make_sc_agg(n_pad, e_pad, d_in)(
      x, src, dst, jnp.zeros((n_pad, d_in), jnp.float32))
  cnt = _make_sc_cnt(n_pad, e_pad)(
      dst, jnp.zeros((n_pad, 128), jnp.float32),
      jnp.ones((_CHUNK, 128), jnp.float32))

  blk = 1000
  grid = (n // blk,)
  full = lambda shape: pl.BlockSpec(shape, lambda i: (0,) * len(shape))
  p, q, invb = pl.pallas_call(
      _tc_layer1,
      grid=grid,
      in_specs=[
          pl.BlockSpec((_NC, blk, d_in), lambda i: (0, i, 0)),
          pl.BlockSpec((_NC, blk, 128), lambda i: (0, i, 0)),
          pl.BlockSpec((blk, d_in), lambda i: (i, 0)),
          full((d_hid, d_in)),
          full((1, d_hid)),
          full((d_hid, d_in)),
          full((d_out, d_hid)),
          full((d_out, d_hid)),
          full((1, d_out)),
      ],
      out_specs=[
          pl.BlockSpec((blk, d_out), lambda i: (i, 0)),
          pl.BlockSpec((blk, d_out), lambda i: (i, 0)),
          pl.BlockSpec((blk, d_out), lambda i: (i, 0)),
      ],
      out_shape=[
          jax.ShapeDtypeStruct((n, d_out), jnp.float32),
          jax.ShapeDtypeStruct((n, d_out), jnp.float32),
          jax.ShapeDtypeStruct((n, d_out), jnp.float32),
      ],
  )(acc1, cnt, x, W1_l, b1.reshape(1, -1), W1_r, W2_l, W2_r, b2.reshape(1, -1))

  acc2 = _make_sc_agg(n_pad, e_pad, d_out)(
      p, src, dst, jnp.zeros((n_pad, d_out), jnp.float32))

  out = pl.pallas_call(
      _tc_final,
      grid=grid,
      in_specs=[
          pl.BlockSpec((_NC, blk, d_out), lambda i: (0, i, 0)),
          pl.BlockSpec((blk, d_out), lambda i: (i, 0)),
          pl.BlockSpec((blk, d_out), lambda i: (i, 0)),
      ],
      out_specs=pl.BlockSpec((blk, d_out), lambda i: (i, 0)),
      out_shape=jax.ShapeDtypeStruct((n, d_out), jnp.float32),
  )(acc2, invb, q)
  return out
